# Initial kernel scaffold; baseline (speedup 1.0000x reference)
import jax
import jax.numpy as jnp
from jax.experimental import pallas as pl


def kernel(x, edge_index, Wl1, Wr1, b1, Wl2, Wr2, b2, Wfc, bfc):
    raise NotImplementedError("write your pallas kernel here")


# speedup vs baseline: 9.0102x; 9.0102x over previous
import functools

import jax
import jax.numpy as jnp
from jax import lax
from jax.experimental import pallas as pl
from jax.experimental.pallas import tpu as pltpu
from jax.experimental.pallas import tpu_sc as plsc

N = 10000
E = 320000
INC = 128
HID = 32
NHH = 4096

NC, NS, L = 2, 16, 16
NW = NC * NS
CH = 128
CPW = -(-E // (NW * CH))
EPW = CPW * CH
E_PAD = NW * EPW
N_PAD = 10112
RPT = N_PAD // NS


def _proj_body(x_ref, wl_ref, wr_ref, p_ref, r_ref):
    x = x_ref[...]
    p_ref[...] = jnp.dot(x, wl_ref[...], preferred_element_type=jnp.float32)
    r_ref[...] = jnp.dot(x, wr_ref[...], preferred_element_type=jnp.float32)


def _proj(x, Wl, Wr, br):
    n, d = x.shape
    return pl.pallas_call(
        _proj_body,
        grid=(n // br,),
        in_specs=[
            pl.BlockSpec((br, d), lambda i: (i, 0)),
            pl.BlockSpec(Wl.shape, lambda i: (0, 0)),
            pl.BlockSpec(Wr.shape, lambda i: (0, 0)),
        ],
        out_specs=[
            pl.BlockSpec((br, HID), lambda i: (i, 0)),
            pl.BlockSpec((br, HID), lambda i: (i, 0)),
        ],
        out_shape=[jax.ShapeDtypeStruct((n, HID), jnp.float32)] * 2,
    )(x, Wl, Wr)


def _mid_body(a0_ref, a1_ref, d0_ref, d1_ref, r_ref, b_ref, wl_ref, wr_ref,
              p_ref, r2_ref):
    deg = jnp.maximum(d0_ref[:, 0:1] + d1_ref[:, 0:1], 1.0)
    h = jnp.maximum(
        (a0_ref[...] + a1_ref[...]) / deg + r_ref[...] + b_ref[0:1, :], 0.0)
    p_ref[...] = jnp.dot(h, wl_ref[...], preferred_element_type=jnp.float32)
    r2_ref[...] = jnp.dot(h, wr_ref[...], preferred_element_type=jnp.float32)


def _mid(a0, a1, d0, d1, r1, b1, Wl2, Wr2, br):
    n = a0.shape[0]
    row = lambda i: (i, 0)
    whole = lambda i: (0, 0)
    return pl.pallas_call(
        _mid_body,
        grid=(n // br,),
        in_specs=[
            pl.BlockSpec((br, HID), row),
            pl.BlockSpec((br, HID), row),
            pl.BlockSpec((br, L), row),
            pl.BlockSpec((br, L), row),
            pl.BlockSpec((br, HID), row),
            pl.BlockSpec((8, HID), whole),
            pl.BlockSpec((HID, HID), whole),
            pl.BlockSpec((HID, HID), whole),
        ],
        out_specs=[
            pl.BlockSpec((br, HID), row),
            pl.BlockSpec((br, HID), row),
        ],
        out_shape=[jax.ShapeDtypeStruct((n, HID), jnp.float32)] * 2,
    )(a0, a1, d0, d1, r1, b1, Wl2, Wr2)


def _head_body(a0_ref, a1_ref, d0_ref, d1_ref, r_ref, b_ref, wfc_ref, bfc_ref,
               o_ref):
    deg = jnp.maximum(d0_ref[:, 0:1] + d1_ref[:, 0:1], 1.0)
    h = jnp.maximum(
        (a0_ref[...] + a1_ref[...]) / deg + r_ref[...] + b_ref[0:1, :], 0.0)
    o_ref[...] = (jnp.dot(h, wfc_ref[...], preferred_element_type=jnp.float32)
                  + bfc_ref[0:1, :])


def _head(a0, a1, d0, d1, r2, b2, Wfc, bfc, br):
    n = a0.shape[0]
    row = lambda i: (i, 0)
    whole = lambda i: (0, 0)
    return pl.pallas_call(
        _head_body,
        grid=(n // br,),
        in_specs=[
            pl.BlockSpec((br, HID), row),
            pl.BlockSpec((br, HID), row),
            pl.BlockSpec((br, L), row),
            pl.BlockSpec((br, L), row),
            pl.BlockSpec((br, HID), row),
            pl.BlockSpec((8, HID), whole),
            pl.BlockSpec((HID, NHH), whole),
            pl.BlockSpec((8, NHH), whole),
        ],
        out_specs=pl.BlockSpec((br, NHH), row),
        out_shape=jax.ShapeDtypeStruct((n, NHH), jnp.float32),
    )(a0, a1, d0, d1, r2, b2, Wfc, bfc)


def _agg_deg_body(p_hbm, src_hbm, dst_hbm, za_hbm, zd_hbm, ones_hbm,
                  agg_out, deg_out,
                  src_v, dst_v, rows_v, ones_v, agg_sh, deg_sh, sem):
    c = lax.axis_index("c")
    s = lax.axis_index("s")
    wid = s * NC + c

    @pl.when(s == 0)
    def _zero():
        pltpu.sync_copy(za_hbm, agg_sh)
        pltpu.sync_copy(zd_hbm, deg_sh)

    pltpu.sync_copy(src_hbm.at[wid], src_v)
    pltpu.sync_copy(dst_hbm.at[wid], dst_v)
    pltpu.sync_copy(ones_hbm, ones_v)
    plsc.subcore_barrier()

    def step(j, carry):
        pltpu.async_copy(p_hbm.at[src_v.at[j]], rows_v, sem).wait()
        pltpu.sync_copy(rows_v, agg_sh.at[dst_v.at[j]], add=True)
        pltpu.sync_copy(ones_v, deg_sh.at[dst_v.at[j]], add=True)
        return carry

    lax.fori_loop(0, CPW, step, 0)
    plsc.subcore_barrier()
    pltpu.sync_copy(agg_sh.at[pl.ds(s * RPT, RPT)],
                    agg_out.at[c, pl.ds(s * RPT, RPT)])
    pltpu.sync_copy(deg_sh.at[pl.ds(s * RPT, RPT)],
                    deg_out.at[c, pl.ds(s * RPT, RPT)])


def _agg_body(p_hbm, src_hbm, dst_hbm, za_hbm,
              agg_out,
              src_v, dst_v, rows_v, agg_sh, sem):
    c = lax.axis_index("c")
    s = lax.axis_index("s")
    wid = s * NC + c

    @pl.when(s == 0)
    def _zero():
        pltpu.sync_copy(za_hbm, agg_sh)

    pltpu.sync_copy(src_hbm.at[wid], src_v)
    pltpu.sync_copy(dst_hbm.at[wid], dst_v)
    plsc.subcore_barrier()

    def step(j, carry):
        pltpu.async_copy(p_hbm.at[src_v.at[j]], rows_v, sem).wait()
        pltpu.sync_copy(rows_v, agg_sh.at[dst_v.at[j]], add=True)
        return carry

    lax.fori_loop(0, CPW, step, 0)
    plsc.subcore_barrier()
    pltpu.sync_copy(agg_sh.at[pl.ds(s * RPT, RPT)],
                    agg_out.at[c, pl.ds(s * RPT, RPT)])


def _edge_agg_deg(p, src3, dst3, za, zd, ones):
    mesh = plsc.VectorSubcoreMesh(core_axis_name="c", subcore_axis_name="s")
    f = pl.kernel(
        _agg_deg_body,
        out_type=[
            jax.ShapeDtypeStruct((NC, N_PAD, HID), jnp.float32),
            jax.ShapeDtypeStruct((NC, N_PAD, L), jnp.float32),
        ],
        mesh=mesh,
        scratch_types=[
            pltpu.VMEM((CPW, CH), jnp.int32),
            pltpu.VMEM((CPW, CH), jnp.int32),
            pltpu.VMEM((CH, HID), jnp.float32),
            pltpu.VMEM((CH, L), jnp.float32),
            pltpu.VMEM_SHARED((N_PAD, HID), jnp.float32),
            pltpu.VMEM_SHARED((N_PAD, L), jnp.float32),
            pltpu.SemaphoreType.DMA,
        ],
        compiler_params=pltpu.CompilerParams(use_tc_tiling_on_sc=False),
    )
    return f(p, src3, dst3, za, zd, ones)


def _edge_agg(p, src3, dst3, za):
    mesh = plsc.VectorSubcoreMesh(core_axis_name="c", subcore_axis_name="s")
    f = pl.kernel(
        _agg_body,
        out_type=jax.ShapeDtypeStruct((NC, N_PAD, HID), jnp.float32),
        mesh=mesh,
        scratch_types=[
            pltpu.VMEM((CPW, CH), jnp.int32),
            pltpu.VMEM((CPW, CH), jnp.int32),
            pltpu.VMEM((CH, HID), jnp.float32),
            pltpu.VMEM_SHARED((N_PAD, HID), jnp.float32),
            pltpu.SemaphoreType.DMA,
        ],
        compiler_params=pltpu.CompilerParams(use_tc_tiling_on_sc=False),
    )
    return f(p, src3, dst3, za)


def kernel(x, edge_index, Wl1, Wr1, b1, Wl2, Wr2, b2, Wfc, bfc):
    src = edge_index[0].astype(jnp.int32)
    dst = edge_index[1].astype(jnp.int32)
    pad = E_PAD - E
    src3 = jnp.concatenate([src, jnp.zeros((pad,), jnp.int32)]).reshape(
        NW, CPW, CH)
    dst3 = jnp.concatenate([dst, jnp.full((pad,), N, jnp.int32)]).reshape(
        NW, CPW, CH)
    za = jnp.zeros((N_PAD, HID), jnp.float32)
    zd = jnp.zeros((N_PAD, L), jnp.float32)
    ones = jnp.ones((CH, L), jnp.float32)
    b1t = jnp.broadcast_to(b1[None, :], (8, HID))
    b2t = jnp.broadcast_to(b2[None, :], (8, HID))
    bfct = jnp.broadcast_to(bfc[None, :], (8, NHH))

    p1, r1 = _proj(x, Wl1, Wr1, 1000)
    agg1, deg = _edge_agg_deg(p1, src3, dst3, za, zd, ones)
    d0, d1 = deg[0, :N], deg[1, :N]
    p2, r2 = _mid(agg1[0, :N], agg1[1, :N], d0, d1, r1, b1t, Wl2, Wr2, 1000)
    agg2 = _edge_agg(p2, src3, dst3, za)
    out = _head(agg2[0, :N], agg2[1, :N], d0, d1, r2, b2t, Wfc, bfct, 400)
    return out


# speedup vs baseline: 9.5312x; 1.0578x over previous
import functools

import jax
import jax.numpy as jnp
from jax import lax
from jax.experimental import pallas as pl
from jax.experimental.pallas import tpu as pltpu
from jax.experimental.pallas import tpu_sc as plsc

N = 10000
E = 320000
INC = 128
HID = 32
NHH = 4096

NC, NS, L = 2, 16, 16
NW = NC * NS
CH = 128
NBUF = 4
CPW = 80
EPW = CPW * CH
E_PAD = NW * EPW
N_PAD = 10112
RPT = N_PAD // NS


def _proj_body(x_ref, wl_ref, wr_ref, p_ref, r_ref):
    x = x_ref[...]
    p_ref[...] = jnp.dot(x, wl_ref[...], preferred_element_type=jnp.float32)
    r_ref[...] = jnp.dot(x, wr_ref[...], preferred_element_type=jnp.float32)


def _proj(x, Wl, Wr, br):
    n, d = x.shape
    return pl.pallas_call(
        _proj_body,
        grid=(n // br,),
        in_specs=[
            pl.BlockSpec((br, d), lambda i: (i, 0)),
            pl.BlockSpec(Wl.shape, lambda i: (0, 0)),
            pl.BlockSpec(Wr.shape, lambda i: (0, 0)),
        ],
        out_specs=[
            pl.BlockSpec((br, HID), lambda i: (i, 0)),
            pl.BlockSpec((br, HID), lambda i: (i, 0)),
        ],
        out_shape=[jax.ShapeDtypeStruct((n, HID), jnp.float32)] * 2,
    )(x, Wl, Wr)


def _mid_body(a0_ref, a1_ref, d0_ref, d1_ref, r_ref, b_ref, wl_ref, wr_ref,
              p_ref, r2_ref):
    deg = jnp.maximum(d0_ref[:, 0:1] + d1_ref[:, 0:1], 1.0)
    h = jnp.maximum(
        (a0_ref[...] + a1_ref[...]) / deg + r_ref[...] + b_ref[0:1, :], 0.0)
    p_ref[...] = jnp.dot(h, wl_ref[...], preferred_element_type=jnp.float32)
    r2_ref[...] = jnp.dot(h, wr_ref[...], preferred_element_type=jnp.float32)


def _mid(a0, a1, d0, d1, r1, b1, Wl2, Wr2, br):
    n = a0.shape[0]
    row = lambda i: (i, 0)
    whole = lambda i: (0, 0)
    return pl.pallas_call(
        _mid_body,
        grid=(n // br,),
        in_specs=[
            pl.BlockSpec((br, HID), row),
            pl.BlockSpec((br, HID), row),
            pl.BlockSpec((br, L), row),
            pl.BlockSpec((br, L), row),
            pl.BlockSpec((br, HID), row),
            pl.BlockSpec((8, HID), whole),
            pl.BlockSpec((HID, HID), whole),
            pl.BlockSpec((HID, HID), whole),
        ],
        out_specs=[
            pl.BlockSpec((br, HID), row),
            pl.BlockSpec((br, HID), row),
        ],
        out_shape=[jax.ShapeDtypeStruct((n, HID), jnp.float32)] * 2,
    )(a0, a1, d0, d1, r1, b1, Wl2, Wr2)


def _head_body(a0_ref, a1_ref, d0_ref, d1_ref, r_ref, b_ref, wfc_ref, bfc_ref,
               o_ref):
    deg = jnp.maximum(d0_ref[:, 0:1] + d1_ref[:, 0:1], 1.0)
    h = jnp.maximum(
        (a0_ref[...] + a1_ref[...]) / deg + r_ref[...] + b_ref[0:1, :], 0.0)
    o_ref[...] = (jnp.dot(h, wfc_ref[...], preferred_element_type=jnp.float32)
                  + bfc_ref[0:1, :])


def _head(a0, a1, d0, d1, r2, b2, Wfc, bfc, br):
    n = a0.shape[0]
    row = lambda i: (i, 0)
    whole = lambda i: (0, 0)
    return pl.pallas_call(
        _head_body,
        grid=(n // br,),
        in_specs=[
            pl.BlockSpec((br, HID), row),
            pl.BlockSpec((br, HID), row),
            pl.BlockSpec((br, L), row),
            pl.BlockSpec((br, L), row),
            pl.BlockSpec((br, HID), row),
            pl.BlockSpec((8, HID), whole),
            pl.BlockSpec((HID, NHH), whole),
            pl.BlockSpec((8, NHH), whole),
        ],
        out_specs=pl.BlockSpec((br, NHH), row),
        out_shape=jax.ShapeDtypeStruct((n, NHH), jnp.float32),
    )(a0, a1, d0, d1, r2, b2, Wfc, bfc)


def _agg_impl(with_deg, p_hbm, src_hbm, dst_hbm, za_hbm, zd_hbm, ones_hbm,
              agg_out, deg_out, src_v, dst_v, rows_v, ones_v, agg_sh, deg_sh,
              gsems):
    c = lax.axis_index("c")
    s = lax.axis_index("s")
    wid = s * NC + c

    @pl.when(s == 0)
    def _zero():
        pltpu.sync_copy(za_hbm, agg_sh)
        if with_deg:
            pltpu.sync_copy(zd_hbm, deg_sh)

    pltpu.sync_copy(src_hbm.at[wid], src_v)
    pltpu.sync_copy(dst_hbm.at[wid], dst_v)
    if with_deg:
        pltpu.sync_copy(ones_hbm, ones_v)
    plsc.subcore_barrier()

    for b in range(NBUF):
        pltpu.async_copy(p_hbm.at[src_v.at[b]], rows_v.at[b], gsems[b])

    def outer(g, carry):
        for b in range(NBUF):
            j = g * NBUF + b
            pltpu.make_async_copy(
                p_hbm.at[src_v.at[j]], rows_v.at[b], gsems[b]).wait()
            pltpu.sync_copy(rows_v.at[b], agg_sh.at[dst_v.at[j]], add=True)
            if with_deg:
                pltpu.sync_copy(ones_v, deg_sh.at[dst_v.at[j]], add=True)

            @pl.when(j + NBUF < CPW)
            def _next():
                pltpu.async_copy(
                    p_hbm.at[src_v.at[j + NBUF]], rows_v.at[b], gsems[b])
        return carry

    lax.fori_loop(0, CPW // NBUF, outer, 0)
    plsc.subcore_barrier()
    pltpu.sync_copy(agg_sh.at[pl.ds(s * RPT, RPT)],
                    agg_out.at[c, pl.ds(s * RPT, RPT)])
    if with_deg:
        pltpu.sync_copy(deg_sh.at[pl.ds(s * RPT, RPT)],
                        deg_out.at[c, pl.ds(s * RPT, RPT)])


def _agg_deg_body(p_hbm, src_hbm, dst_hbm, za_hbm, zd_hbm, ones_hbm,
                  agg_out, deg_out,
                  src_v, dst_v, rows_v, ones_v, agg_sh, deg_sh,
                  gs0, gs1, gs2, gs3):
    _agg_impl(True, p_hbm, src_hbm, dst_hbm, za_hbm, zd_hbm, ones_hbm,
              agg_out, deg_out, src_v, dst_v, rows_v, ones_v, agg_sh, deg_sh,
              (gs0, gs1, gs2, gs3))


def _agg_body(p_hbm, src_hbm, dst_hbm, za_hbm,
              agg_out,
              src_v, dst_v, rows_v, agg_sh,
              gs0, gs1, gs2, gs3):
    _agg_impl(False, p_hbm, src_hbm, dst_hbm, za_hbm, None, None,
              agg_out, None, src_v, dst_v, rows_v, None, agg_sh, None,
              (gs0, gs1, gs2, gs3))


def _edge_agg_deg(p, src3, dst3, za, zd, ones):
    mesh = plsc.VectorSubcoreMesh(core_axis_name="c", subcore_axis_name="s")
    f = pl.kernel(
        _agg_deg_body,
        out_type=[
            jax.ShapeDtypeStruct((NC, N_PAD, HID), jnp.float32),
            jax.ShapeDtypeStruct((NC, N_PAD, L), jnp.float32),
        ],
        mesh=mesh,
        scratch_types=[
            pltpu.VMEM((CPW, CH), jnp.int32),
            pltpu.VMEM((CPW, CH), jnp.int32),
            pltpu.VMEM((NBUF, CH, HID), jnp.float32),
            pltpu.VMEM((CH, L), jnp.float32),
            pltpu.VMEM_SHARED((N_PAD, HID), jnp.float32),
            pltpu.VMEM_SHARED((N_PAD, L), jnp.float32),
            pltpu.SemaphoreType.DMA,
            pltpu.SemaphoreType.DMA,
            pltpu.SemaphoreType.DMA,
            pltpu.SemaphoreType.DMA,
        ],
        compiler_params=pltpu.CompilerParams(use_tc_tiling_on_sc=False),
    )
    return f(p, src3, dst3, za, zd, ones)


def _edge_agg(p, src3, dst3, za):
    mesh = plsc.VectorSubcoreMesh(core_axis_name="c", subcore_axis_name="s")
    f = pl.kernel(
        _agg_body,
        out_type=jax.ShapeDtypeStruct((NC, N_PAD, HID), jnp.float32),
        mesh=mesh,
        scratch_types=[
            pltpu.VMEM((CPW, CH), jnp.int32),
            pltpu.VMEM((CPW, CH), jnp.int32),
            pltpu.VMEM((NBUF, CH, HID), jnp.float32),
            pltpu.VMEM_SHARED((N_PAD, HID), jnp.float32),
            pltpu.SemaphoreType.DMA,
            pltpu.SemaphoreType.DMA,
            pltpu.SemaphoreType.DMA,
            pltpu.SemaphoreType.DMA,
        ],
        compiler_params=pltpu.CompilerParams(use_tc_tiling_on_sc=False),
    )
    return f(p, src3, dst3, za)


def kernel(x, edge_index, Wl1, Wr1, b1, Wl2, Wr2, b2, Wfc, bfc):
    src = edge_index[0].astype(jnp.int32)
    dst = edge_index[1].astype(jnp.int32)
    pad = E_PAD - E
    src3 = jnp.concatenate([src, jnp.zeros((pad,), jnp.int32)]).reshape(
        NW, CPW, CH)
    dst3 = jnp.concatenate([dst, jnp.full((pad,), N, jnp.int32)]).reshape(
        NW, CPW, CH)
    za = jnp.zeros((N_PAD, HID), jnp.float32)
    zd = jnp.zeros((N_PAD, L), jnp.float32)
    ones = jnp.ones((CH, L), jnp.float32)
    b1t = jnp.broadcast_to(b1[None, :], (8, HID))
    b2t = jnp.broadcast_to(b2[None, :], (8, HID))
    bfct = jnp.broadcast_to(bfc[None, :], (8, NHH))

    p1, r1 = _proj(x, Wl1, Wr1, 1000)
    agg1, deg = _edge_agg_deg(p1, src3, dst3, za, zd, ones)
    d0, d1 = deg[0, :N], deg[1, :N]
    p2, r2 = _mid(agg1[0, :N], agg1[1, :N], d0, d1, r1, b1t, Wl2, Wr2, 1000)
    agg2 = _edge_agg(p2, src3, dst3, za)
    out = _head(agg2[0, :N], agg2[1, :N], d0, d1, r2, b2t, Wfc, bfct, 400)
    return out
